# baseline (speedup 1.0000x reference)
import functools

import jax
import jax.numpy as jnp
from jax.experimental import pallas as pl
from jax.experimental.pallas import tpu as pltpu

EPS = 1e-6
BETA_CE = 1.0
BETA_CU = 1.0


def _mm_kernel(x_ref, w_ref, o_ref, acc_ref, *, nk):
    @pl.when(pl.program_id(2) == 0)
    def _init():
        acc_ref[...] = jnp.zeros_like(acc_ref)

    acc_ref[...] += jnp.dot(x_ref[...], w_ref[...],
                            preferred_element_type=jnp.float32)

    @pl.when(pl.program_id(2) == nk - 1)
    def _done():
        o_ref[...] = acc_ref[...].astype(o_ref.dtype)


def _mm(x, w, bm=1024, bn=1024, bk=512, out_dtype=None):
    m, k = x.shape
    _, n = w.shape
    bm, bn, bk = min(bm, m), min(bn, n), min(bk, k)
    nm, nn, nk = m // bm, n // bn, k // bk
    out_dtype = out_dtype or x.dtype
    return pl.pallas_call(
        functools.partial(_mm_kernel, nk=nk),
        grid=(nm, nn, nk),
        in_specs=[
            pl.BlockSpec((bm, bk), lambda i, j, kk: (i, kk)),
            pl.BlockSpec((bk, bn), lambda i, j, kk: (kk, j)),
        ],
        out_specs=pl.BlockSpec((bm, bn), lambda i, j, kk: (i, j)),
        out_shape=jax.ShapeDtypeStruct((m, n), out_dtype),
        scratch_shapes=[pltpu.VMEM((bm, bn), jnp.float32)],
        compiler_params=pltpu.CompilerParams(
            dimension_semantics=("parallel", "parallel", "arbitrary")),
    )(x, w)


def _glu_kernel(x_ref, wg_ref, wu_ref, h_ref):
    x = x_ref[...]
    g = jnp.dot(x, wg_ref[...], preferred_element_type=jnp.float32)
    u = jnp.dot(x, wu_ref[...], preferred_element_type=jnp.float32)
    h_ref[...] = (g * jax.nn.sigmoid(g) * u).astype(h_ref.dtype)


def _glu(x, wg, wu, bm=1024, bn=512):
    m, k = x.shape
    _, n = wg.shape
    bm, bn = min(bm, m), min(bn, n)
    return pl.pallas_call(
        _glu_kernel,
        grid=(m // bm, n // bn),
        in_specs=[
            pl.BlockSpec((bm, k), lambda i, j: (i, 0)),
            pl.BlockSpec((k, bn), lambda i, j: (0, j)),
            pl.BlockSpec((k, bn), lambda i, j: (0, j)),
        ],
        out_specs=pl.BlockSpec((bm, bn), lambda i, j: (i, j)),
        out_shape=jax.ShapeDtypeStruct((m, n), x.dtype),
        compiler_params=pltpu.CompilerParams(
            dimension_semantics=("parallel", "parallel")),
    )(x, wg, wu)


def _attn_kernel(q_ref, k_ref, v_ref, o_ref, *, bq, bk, scale):
    qi = pl.program_id(1)
    hd = q_ref.shape[1]
    q = q_ref[...]
    nkb = (qi + 1) * (bq // bk)

    def body(kb, carry):
        m, l, acc = carry
        ks = k_ref[pl.ds(kb * bk, bk), :]
        s = jax.lax.dot_general(q, ks, (((1,), (1,)), ((), ())),
                                preferred_element_type=jnp.float32) * scale
        rows = qi * bq + jax.lax.broadcasted_iota(jnp.int32, (bq, bk), 0)
        cols = kb * bk + jax.lax.broadcasted_iota(jnp.int32, (bq, bk), 1)
        s = jnp.where(cols <= rows, s, -1e30)
        m_new = jnp.maximum(m, jnp.max(s, axis=1, keepdims=True))
        alpha = jnp.exp(m - m_new)
        p = jnp.exp(s - m_new)
        l_new = l * alpha + jnp.sum(p, axis=1, keepdims=True)
        vs = v_ref[pl.ds(kb * bk, bk), :]
        acc_new = acc * alpha + jnp.dot(p, vs,
                                        preferred_element_type=jnp.float32)
        return m_new, l_new, acc_new

    m0 = jnp.full((bq, 1), -jnp.inf, jnp.float32)
    l0 = jnp.zeros((bq, 1), jnp.float32)
    acc0 = jnp.zeros((bq, hd), jnp.float32)
    m, l, acc = jax.lax.fori_loop(0, nkb, body, (m0, l0, acc0))
    o_ref[...] = (acc / l).astype(o_ref.dtype)


def _attention(q, k, v, h, hd, bq=256, bk=256):
    t, d = q.shape
    bq = min(bq, t)
    bk = min(bk, t)
    scale = 1.0 / (hd ** 0.5)
    return pl.pallas_call(
        functools.partial(_attn_kernel, bq=bq, bk=bk, scale=scale),
        grid=(h, t // bq),
        in_specs=[
            pl.BlockSpec((bq, hd), lambda hh, qi: (qi, hh)),
            pl.BlockSpec((t, hd), lambda hh, qi: (0, hh)),
            pl.BlockSpec((t, hd), lambda hh, qi: (0, hh)),
        ],
        out_specs=pl.BlockSpec((bq, hd), lambda hh, qi: (qi, hh)),
        out_shape=jax.ShapeDtypeStruct((t, d), jnp.float32),
        compiler_params=pltpu.CompilerParams(
            dimension_semantics=("arbitrary", "arbitrary")),
    )(q, k, v)


def _rms(x, w):
    return x * jax.lax.rsqrt(jnp.mean(x * x, axis=-1, keepdims=True) + EPS) * w


def _rope_cos_sin(t, hd):
    inv = 1.0 / (10000.0 ** (jnp.arange(0, hd, 2, dtype=jnp.float32) / hd))
    freqs = jnp.arange(t, dtype=jnp.float32)[:, None] * inv[None, :]
    cos = jnp.concatenate([jnp.cos(freqs), jnp.cos(freqs)], axis=-1)
    sin = jnp.concatenate([jnp.sin(freqs), jnp.sin(freqs)], axis=-1)
    return cos, sin


def _rope_tD(x, cos, sin, h, hd):
    t, d = x.shape
    xh = x.reshape(t, h, hd)
    x1 = xh[..., :hd // 2]
    x2 = xh[..., hd // 2:]
    rot = jnp.concatenate([-x2, x1], axis=-1)
    out = xh * cos[:, None, :] + rot * sin[:, None, :]
    return out.reshape(t, d)


def _decoder(x, p, pref, h, hd, cos, sin):
    xn = _rms(x, p[pref + 'ln1'])
    q = _mm(xn, p[pref + 'wq'])
    k = _mm(xn, p[pref + 'wk'])
    v = _mm(xn, p[pref + 'wv'])
    q = _rope_tD(q, cos, sin, h, hd)
    k = _rope_tD(k, cos, sin, h, hd)
    ao = _attention(q, k, v, h, hd)
    x = x + _mm(ao, p[pref + 'wo'])
    hn = _rms(x, p[pref + 'ln2'])
    hh = _glu(hn, p[pref + 'wg'], p[pref + 'wu'])
    return x + _mm(hh, p[pref + 'wd'])


def kernel(hidden_states, params):
    p = params
    b, t, d = hidden_states.shape
    x = hidden_states.reshape(t, d)
    h = 16
    hd = d // h
    cos, sin = _rope_cos_sin(t, hd)

    processed = _decoder(x, p, 'l1_', h, hd, cos, sin)
    pn = _rms(x, p['p_ln'])
    ph = _glu(pn, p['p_wg'], p['p_wu'])
    prior_out = _mm(ph, p['p_wd'])
    prior_hidden = x + prior_out

    prior_loss = jnp.mean((prior_hidden - processed) ** 2)

    actual = processed - x
    predicted = prior_out
    D_st = jnp.sum(actual ** 2, axis=-1) / d
    D_ch = jnp.sum((actual - predicted) ** 2, axis=-1) / d
    z_st = (D_st - jnp.mean(D_st)) / (jnp.std(D_st) + 1e-6)
    z_ch = (D_ch - jnp.mean(D_ch)) / (jnp.std(D_ch) + 1e-6)
    g_cont = jax.nn.sigmoid(BETA_CE * z_st - BETA_CU * z_ch)

    kk = max(1, int(t * 0.5))
    gscores, topk_idx = jax.lax.top_k(g_cont, kk)

    binary = jnp.zeros((t,), jnp.float32).at[topk_idx].set(1.0)
    logits = x @ p['r_w']
    causal_loss = jnp.mean(jnp.maximum(logits, 0.0) - logits * binary
                           + jnp.log1p(jnp.exp(-jnp.abs(logits))))

    sel = processed[topk_idx]
    out2 = _decoder(sel, p, 'l2_', h, hd, cos[:kk], sin[:kk])
    new = sel + gscores[:, None] * (out2 - sel)
    final = processed.at[topk_idx].set(new)

    return final.reshape(b, t, d), prior_loss, causal_loss


# speedup vs baseline: 1.2473x; 1.2473x over previous
import functools

import jax
import jax.numpy as jnp
from jax.experimental import pallas as pl
from jax.experimental.pallas import tpu as pltpu

EPS = 1e-6
BETA_CE = 1.0
BETA_CU = 1.0


def _mm_kernel(x_ref, w_ref, o_ref, acc_ref, *, nk):
    @pl.when(pl.program_id(2) == 0)
    def _init():
        acc_ref[...] = jnp.zeros_like(acc_ref)

    acc_ref[...] += jnp.dot(x_ref[...], w_ref[...],
                            preferred_element_type=jnp.float32)

    @pl.when(pl.program_id(2) == nk - 1)
    def _done():
        o_ref[...] = acc_ref[...].astype(o_ref.dtype)


def _mm(x, w, bm=1024, bn=2048, bk=512, out_dtype=None):
    m, k = x.shape
    _, n = w.shape
    bm, bn, bk = min(bm, m), min(bn, n), min(bk, k)
    nm, nn, nk = m // bm, n // bn, k // bk
    out_dtype = out_dtype or x.dtype
    return pl.pallas_call(
        functools.partial(_mm_kernel, nk=nk),
        grid=(nm, nn, nk),
        in_specs=[
            pl.BlockSpec((bm, bk), lambda i, j, kk: (i, kk)),
            pl.BlockSpec((bk, bn), lambda i, j, kk: (kk, j)),
        ],
        out_specs=pl.BlockSpec((bm, bn), lambda i, j, kk: (i, j)),
        out_shape=jax.ShapeDtypeStruct((m, n), out_dtype),
        scratch_shapes=[pltpu.VMEM((bm, bn), jnp.float32)],
        compiler_params=pltpu.CompilerParams(
            dimension_semantics=("parallel", "parallel", "arbitrary")),
    )(x, w)


def _glu_kernel(x_ref, wg_ref, wu_ref, h_ref):
    x = x_ref[...]
    g = jnp.dot(x, wg_ref[...], preferred_element_type=jnp.float32)
    u = jnp.dot(x, wu_ref[...], preferred_element_type=jnp.float32)
    h_ref[...] = (g * jax.nn.sigmoid(g) * u).astype(h_ref.dtype)


def _glu(x, wg, wu, bm=1024, bn=512):
    m, k = x.shape
    _, n = wg.shape
    bm, bn = min(bm, m), min(bn, n)
    return pl.pallas_call(
        _glu_kernel,
        grid=(m // bm, n // bn),
        in_specs=[
            pl.BlockSpec((bm, k), lambda i, j: (i, 0)),
            pl.BlockSpec((k, bn), lambda i, j: (0, j)),
            pl.BlockSpec((k, bn), lambda i, j: (0, j)),
        ],
        out_specs=pl.BlockSpec((bm, bn), lambda i, j: (i, j)),
        out_shape=jax.ShapeDtypeStruct((m, n), x.dtype),
        compiler_params=pltpu.CompilerParams(
            dimension_semantics=("parallel", "parallel")),
    )(x, wg, wu)


def _attn_kernel(q_ref, k_ref, v_ref, o_ref, *, bq, bk, scale):
    qi = pl.program_id(1)
    hd = q_ref.shape[1]
    q = q_ref[...]
    nkb = (qi + 1) * (bq // bk)

    def body(kb, carry):
        m, l, acc = carry
        ks = k_ref[pl.ds(kb * bk, bk), :]
        s = jax.lax.dot_general(q, ks, (((1,), (1,)), ((), ())),
                                preferred_element_type=jnp.float32) * scale
        rows = qi * bq + jax.lax.broadcasted_iota(jnp.int32, (bq, bk), 0)
        cols = kb * bk + jax.lax.broadcasted_iota(jnp.int32, (bq, bk), 1)
        s = jnp.where(cols <= rows, s, -1e30)
        m_new = jnp.maximum(m, jnp.max(s, axis=1, keepdims=True))
        alpha = jnp.exp(m - m_new)
        p = jnp.exp(s - m_new)
        l_new = l * alpha + jnp.sum(p, axis=1, keepdims=True)
        vs = v_ref[pl.ds(kb * bk, bk), :]
        acc_new = acc * alpha + jnp.dot(p, vs,
                                        preferred_element_type=jnp.float32)
        return m_new, l_new, acc_new

    m0 = jnp.full((bq, 1), -jnp.inf, jnp.float32)
    l0 = jnp.zeros((bq, 1), jnp.float32)
    acc0 = jnp.zeros((bq, hd), jnp.float32)
    m, l, acc = jax.lax.fori_loop(0, nkb, body, (m0, l0, acc0))
    o_ref[...] = (acc / l).astype(o_ref.dtype)


def _attention(q, k, v, h, hd, bq=1024, bk=1024):
    t, d = q.shape
    bq = min(bq, t)
    bk = min(bk, t)
    scale = 1.0 / (hd ** 0.5)
    return pl.pallas_call(
        functools.partial(_attn_kernel, bq=bq, bk=bk, scale=scale),
        grid=(h, t // bq),
        in_specs=[
            pl.BlockSpec((bq, hd), lambda hh, qi: (qi, hh)),
            pl.BlockSpec((t, hd), lambda hh, qi: (0, hh)),
            pl.BlockSpec((t, hd), lambda hh, qi: (0, hh)),
        ],
        out_specs=pl.BlockSpec((bq, hd), lambda hh, qi: (qi, hh)),
        out_shape=jax.ShapeDtypeStruct((t, d), jnp.float32),
        compiler_params=pltpu.CompilerParams(
            dimension_semantics=("arbitrary", "arbitrary")),
    )(q, k, v)


def _rms(x, w):
    return x * jax.lax.rsqrt(jnp.mean(x * x, axis=-1, keepdims=True) + EPS) * w


def _rope_cos_sin(t, hd):
    inv = 1.0 / (10000.0 ** (jnp.arange(0, hd, 2, dtype=jnp.float32) / hd))
    freqs = jnp.arange(t, dtype=jnp.float32)[:, None] * inv[None, :]
    cos = jnp.concatenate([jnp.cos(freqs), jnp.cos(freqs)], axis=-1)
    sin = jnp.concatenate([jnp.sin(freqs), jnp.sin(freqs)], axis=-1)
    return cos, sin


def _rope_tD(x, cos, sin, h, hd):
    t, d = x.shape
    xh = x.reshape(t, h, hd)
    x1 = xh[..., :hd // 2]
    x2 = xh[..., hd // 2:]
    rot = jnp.concatenate([-x2, x1], axis=-1)
    out = xh * cos[:, None, :] + rot * sin[:, None, :]
    return out.reshape(t, d)


def _decoder(x, p, pref, h, hd, cos, sin):
    xn = _rms(x, p[pref + 'ln1'])
    q = _mm(xn, p[pref + 'wq'])
    k = _mm(xn, p[pref + 'wk'])
    v = _mm(xn, p[pref + 'wv'])
    q = _rope_tD(q, cos, sin, h, hd)
    k = _rope_tD(k, cos, sin, h, hd)
    ao = _attention(q, k, v, h, hd)
    x = x + _mm(ao, p[pref + 'wo'])
    hn = _rms(x, p[pref + 'ln2'])
    hh = _glu(hn, p[pref + 'wg'], p[pref + 'wu'])
    return x + _mm(hh, p[pref + 'wd'])


def kernel(hidden_states, params):
    p = params
    b, t, d = hidden_states.shape
    x = hidden_states.reshape(t, d)
    h = 16
    hd = d // h
    cos, sin = _rope_cos_sin(t, hd)

    processed = _decoder(x, p, 'l1_', h, hd, cos, sin)
    pn = _rms(x, p['p_ln'])
    ph = _glu(pn, p['p_wg'], p['p_wu'])
    prior_out = _mm(ph, p['p_wd'])
    prior_hidden = x + prior_out

    prior_loss = jnp.mean((prior_hidden - processed) ** 2)

    actual = processed - x
    predicted = prior_out
    D_st = jnp.sum(actual ** 2, axis=-1) / d
    D_ch = jnp.sum((actual - predicted) ** 2, axis=-1) / d
    z_st = (D_st - jnp.mean(D_st)) / (jnp.std(D_st) + 1e-6)
    z_ch = (D_ch - jnp.mean(D_ch)) / (jnp.std(D_ch) + 1e-6)
    g_cont = jax.nn.sigmoid(BETA_CE * z_st - BETA_CU * z_ch)

    kk = max(1, int(t * 0.5))
    gscores, topk_idx = jax.lax.top_k(g_cont, kk)

    binary = jnp.zeros((t,), jnp.float32).at[topk_idx].set(1.0)
    logits = x @ p['r_w']
    causal_loss = jnp.mean(jnp.maximum(logits, 0.0) - logits * binary
                           + jnp.log1p(jnp.exp(-jnp.abs(logits))))

    sel = processed[topk_idx]
    out2 = _decoder(sel, p, 'l2_', h, hd, cos[:kk], sin[:kk])
    new = sel + gscores[:, None] * (out2 - sel)
    final = processed.at[topk_idx].set(new)

    return final.reshape(b, t, d), prior_loss, causal_loss
